# baseline (speedup 1.0000x reference)
import functools

import jax
import jax.numpy as jnp
from jax.experimental import pallas as pl
from jax.experimental.pallas import tpu as pltpu

L = 2048
D = 1024
NH = 16
HD = 64
DC = 128
DFF = 2048
NE = 8
EPS = 1.1920929e-07
NEG = -1e30
BF = jnp.bfloat16
F32 = jnp.float32
I32 = jnp.int32
BT = 256
NT = (2 * L) // BT + NE
PADT = NT * BT
HIGHEST = jax.lax.Precision.HIGHEST


def _dot(a, b):
    return jax.lax.dot_general(a.astype(BF), b.astype(BF),
                               (((a.ndim - 1,), (0,)), ((), ())),
                               preferred_element_type=F32)


def _dot_t(a, b):
    return jax.lax.dot_general(a.astype(BF), b.astype(BF),
                               (((1,), (1,)), ((), ())),
                               preferred_element_type=F32)


def _dot_f32(a, b):
    return jax.lax.dot_general(a, b, (((a.ndim - 1,), (0,)), ((), ())),
                               precision=HIGHEST,
                               preferred_element_type=F32)


def _qkv_body(x_ref, anw_ref, wkv_ref, wkc_ref, wvc_ref, wqr_ref, wkr_ref,
              q_ref, kc_ref, kr_ref, v_ref):
    x = x_ref[...]
    var = jnp.mean(x * x, axis=-1, keepdims=True)
    h = x * jax.lax.rsqrt(var + EPS) * anw_ref[...]
    scale = HD ** -0.5
    q_ref[...] = (_dot(h, wqr_ref[...]) * scale).astype(BF)
    c = _dot(h, wkv_ref[...])
    kc_ref[...] = _dot(c, wkc_ref[...]).astype(BF)
    kr_ref[...] = _dot(h, wkr_ref[...]).astype(BF)
    v_ref[...] = _dot(c, wvc_ref[...]).astype(BF)


def _qkv(x2d, anw, wkv, wkc, wvc, wqr, wkr):
    blk = 512
    w_spec = lambda shape: pl.BlockSpec(shape, lambda i: (0,) * len(shape))
    row = pl.BlockSpec((blk, D), lambda i: (i, 0))
    return pl.pallas_call(
        _qkv_body,
        grid=(L // blk,),
        in_specs=[row, w_spec((1, D)), w_spec((D, DC)), w_spec((DC, D)),
                  w_spec((DC, D)), w_spec((D, D)), w_spec((D, D))],
        out_specs=[row, row, row, row],
        out_shape=[jax.ShapeDtypeStruct((L, D), BF)] * 4,
    )(x2d, anw.reshape(1, D), wkv, wkc, wvc, wqr, wkr)


def _attn_body(q_ref, kc_ref, kr_ref, v_ref, o_ref):
    cb = 512
    for hh in range(2):
        sl = slice(hh * HD, (hh + 1) * HD)
        kc = kc_ref[:, sl]
        kr = kr_ref[:, sl]
        v = v_ref[:, sl]
        for c0 in range(0, L, cb):
            q = q_ref[c0:c0 + cb, sl]
            s = _dot_t(q, kc) + _dot_t(q, kr)
            m = jnp.max(s, axis=-1, keepdims=True)
            p = jnp.exp(s - m)
            denom = jnp.sum(p, axis=-1, keepdims=True)
            o = _dot((p / denom).astype(BF), v)
            o_ref[c0:c0 + cb, sl] = o.astype(BF)


def _attention(q, kc, kr, v):
    pair = pl.BlockSpec((L, 2 * HD), lambda i: (0, i))
    return pl.pallas_call(
        _attn_body,
        grid=(NH // 2,),
        in_specs=[pair, pair, pair, pair],
        out_specs=pair,
        out_shape=jax.ShapeDtypeStruct((L, D), BF),
    )(q, kc, kr, v)


def _post_body(attn_ref, x_ref, wo_ref, fnw_ref, gw_ref, gb_ref,
               x2_ref, h2_ref, e1_ref, e2_ref, tw1_ref, tw2_ref, cnt_ref):
    i = pl.program_id(0)
    x2 = _dot(attn_ref[...], wo_ref[...]) + x_ref[...]
    x2_ref[...] = x2
    var = jnp.mean(x2 * x2, axis=-1, keepdims=True)
    h2 = x2 * jax.lax.rsqrt(var + EPS) * fnw_ref[...]
    h2b = h2.astype(BF)
    h2_ref[...] = h2b
    logits = _dot(h2b, gw_ref[...]) + gb_ref[...]
    rows = logits.shape[0]
    iota = jax.lax.broadcasted_iota(I32, (rows, NE), 1)
    m1 = jnp.max(logits, axis=-1, keepdims=True)
    i1 = jnp.min(jnp.where(logits == m1, iota, NE), axis=-1, keepdims=True)
    l2 = jnp.where(iota == i1, NEG, logits)
    m2 = jnp.max(l2, axis=-1, keepdims=True)
    i2 = jnp.min(jnp.where(l2 == m2, iota, NE), axis=-1, keepdims=True)
    tw1 = 1.0 / (1.0 + jnp.exp(m2 - m1))
    e1_ref[...] = i1
    e2_ref[...] = i2
    tw1_ref[...] = tw1
    tw2_ref[...] = 1.0 - tw1
    oh = ((iota == i1) | (iota == i2)).astype(F32)
    cnt = jnp.sum(oh, axis=0, keepdims=True)

    @pl.when(i == 0)
    def _():
        cnt_ref[...] = jnp.zeros_like(cnt_ref)

    cnt_ref[...] += cnt


def _post(attn, x2d, wo, fnw, gw, gb):
    blk = 512
    w_spec = lambda shape: pl.BlockSpec(shape, lambda i: (0,) * len(shape))
    row = pl.BlockSpec((blk, D), lambda i: (i, 0))
    col = pl.BlockSpec((blk, 1), lambda i: (i, 0))
    return pl.pallas_call(
        _post_body,
        grid=(L // blk,),
        in_specs=[row, row, w_spec((D, D)), w_spec((1, D)), w_spec((D, NE)),
                  w_spec((1, NE))],
        out_specs=[row, row, col, col, col, col, w_spec((1, NE))],
        out_shape=[
            jax.ShapeDtypeStruct((L, D), F32),
            jax.ShapeDtypeStruct((L, D), BF),
            jax.ShapeDtypeStruct((L, 1), I32),
            jax.ShapeDtypeStruct((L, 1), I32),
            jax.ShapeDtypeStruct((L, 1), F32),
            jax.ShapeDtypeStruct((L, 1), F32),
            jax.ShapeDtypeStruct((1, NE), F32),
        ],
    )(attn, x2d, wo, fnw.reshape(1, D), gw, gb.reshape(1, NE))


def _meta_body(e1_ref, e2_ref, pos1_ref, pos2_ref, sc_ref):
    iota8 = jax.lax.broadcasted_iota(I32, (1, NE), 1)
    oh1 = (e1_ref[...] == iota8).astype(F32)
    oh2 = (e2_ref[...] == iota8).astype(F32)
    ch = 256
    ir = jax.lax.broadcasted_iota(I32, (ch, ch), 0)
    ic = jax.lax.broadcasted_iota(I32, (ch, ch), 1)
    tstrict = (ic < ir).astype(BF)
    carry = jnp.zeros((1, NE), F32)
    ranks = []
    for oh in (oh1, oh2):
        rs = []
        for c0 in range(0, L, ch):
            blk = oh[c0:c0 + ch, :]
            part = jax.lax.dot_general(
                tstrict, blk.astype(BF), (((1,), (0,)), ((), ())),
                preferred_element_type=F32)
            rs.append(part + carry)
            carry = carry + jnp.sum(blk, axis=0, keepdims=True)
        ranks.append(jnp.concatenate(rs, axis=0))
    counts = carry
    padded = jnp.ceil(counts / BT) * BT
    e8r = jax.lax.broadcasted_iota(I32, (NE, NE), 0)
    e8c = jax.lax.broadcasted_iota(I32, (NE, NE), 1)
    mupper = (e8r < e8c).astype(F32)
    off = _dot_f32(padded, mupper)
    pos1_ref[...] = jnp.sum(oh1 * (off + ranks[0]), axis=-1,
                            keepdims=True).astype(I32)
    pos2_ref[...] = jnp.sum(oh2 * (off + ranks[1]), axis=-1,
                            keepdims=True).astype(I32)
    tile_start = off / BT
    n_used = jnp.sum(padded) / BT
    jcol = jax.lax.broadcasted_iota(I32, (NT, 1), 0).astype(F32)
    jclamp = jnp.minimum(jcol, n_used - 1.0)
    emap = jnp.sum((tile_start <= jclamp).astype(F32), axis=-1,
                   keepdims=True) - 1.0
    active = (jcol < n_used).astype(F32)
    sc_ref[...] = jnp.concatenate([emap, active], axis=1).astype(I32)


def _meta(e1, e2):
    full = lambda shape: pl.BlockSpec(shape, lambda: (0,) * len(shape))
    return pl.pallas_call(
        _meta_body,
        in_specs=[full((L, 1)), full((L, 1))],
        out_specs=[full((L, 1)), full((L, 1)), full((NT, 2))],
        out_shape=[
            jax.ShapeDtypeStruct((L, 1), I32),
            jax.ShapeDtypeStruct((L, 1), I32),
            jax.ShapeDtypeStruct((NT, 2), I32),
        ],
    )(e1, e2)


def _silu(a):
    return a * (0.5 * (jnp.tanh(a * 0.5) + 1.0))


def _gmlp_body(sc_ref, posr_ref, twc_ref, h2_ref, w1a_ref, w1b_ref, w2_ref,
               es_ref):
    j = pl.program_id(0)
    base = j * BT

    @pl.when(sc_ref[j, 1] == 0)
    def _():
        es_ref[...] = jnp.zeros_like(es_ref)

    @pl.when(sc_ref[j, 1] == 1)
    def _():
        iota_r = jax.lax.broadcasted_iota(I32, (BT, 1), 0) + base
        a_sel = (iota_r == posr_ref[...]).astype(F32)
        pair_tok = jax.lax.broadcasted_iota(I32, (2 * L, 1), 0)
        pair_tok = jnp.where(pair_tok >= L, pair_tok - L, pair_tok)
        row_ids = _dot_f32(a_sel, pair_tok.astype(F32))
        ws = _dot_f32(a_sel, twc_ref[...])
        iota_c = jax.lax.broadcasted_iota(I32, (BT, L), 1).astype(F32)
        g_sel = (row_ids == iota_c).astype(BF)
        xs = jax.lax.dot_general(g_sel, h2_ref[...],
                                 (((1,), (0,)), ((), ())),
                                 preferred_element_type=F32)
        xsb = xs.astype(BF)
        a = _dot(xsb, w1a_ref[0])
        b = _dot(xsb, w1b_ref[0])
        g = (_silu(a) * b).astype(BF)
        eo = _dot(g, w2_ref[0])
        es_ref[...] = (eo * ws).astype(BF)


def _gmlp(sc, pos_row, tw_col, h2b, w1a, w1b, w2):
    grid_spec = pltpu.PrefetchScalarGridSpec(
        num_scalar_prefetch=1,
        grid=(NT,),
        in_specs=[
            pl.BlockSpec((1, 2 * L), lambda j, sc: (0, 0)),
            pl.BlockSpec((2 * L, 1), lambda j, sc: (0, 0)),
            pl.BlockSpec((L, D), lambda j, sc: (0, 0)),
            pl.BlockSpec((1, D, DFF), lambda j, sc: (sc[j, 0], 0, 0)),
            pl.BlockSpec((1, D, DFF), lambda j, sc: (sc[j, 0], 0, 0)),
            pl.BlockSpec((1, DFF, D), lambda j, sc: (sc[j, 0], 0, 0)),
        ],
        out_specs=pl.BlockSpec((BT, D), lambda j, sc: (j, 0)),
    )
    return pl.pallas_call(
        _gmlp_body,
        grid_spec=grid_spec,
        out_shape=jax.ShapeDtypeStruct((PADT, D), BF),
    )(sc, pos_row, tw_col, h2b, w1a, w1b, w2)


def _combine_body(pos1_ref, pos2_ref, x2_ref, es_ref, out_ref):
    iota_c = jax.lax.broadcasted_iota(I32, (BT, PADT), 1)
    w_sel = ((iota_c == pos1_ref[...]).astype(BF)
             + (iota_c == pos2_ref[...]).astype(BF))
    moe = jax.lax.dot_general(w_sel, es_ref[...], (((1,), (0,)), ((), ())),
                              preferred_element_type=F32)
    out_ref[...] = moe + x2_ref[...]


def _combine(pos1, pos2, x2, es):
    col = pl.BlockSpec((BT, 1), lambda i: (i, 0))
    row = pl.BlockSpec((BT, D), lambda i: (i, 0))
    full = lambda shape: pl.BlockSpec(shape, lambda i: (0,) * len(shape))
    return pl.pallas_call(
        _combine_body,
        grid=(L // BT,),
        in_specs=[col, col, row, full((PADT, D))],
        out_specs=row,
        out_shape=jax.ShapeDtypeStruct((L, D), F32),
    )(pos1, pos2, x2, es)


def kernel(x, attn_norm_w, ffn_norm_w, w_kv_c, w_kc_up, w_vc_up, w_qr, w_kr,
           w_o, gate_w, expert_bias, expert_w1, expert_w2):
    x2d = x.reshape(L, D)
    q, kc, kr, v = _qkv(x2d, attn_norm_w, w_kv_c, w_kc_up, w_vc_up, w_qr,
                        w_kr)
    attn = _attention(q, kc, kr, v)
    x2, h2b, e1, e2, tw1, tw2, cnt = _post(attn, x2d, w_o, ffn_norm_w,
                                           gate_w, expert_bias)
    pos1, pos2, sc = _meta(e1, e2)
    pos_row = jnp.concatenate([pos1, pos2], axis=0).reshape(1, 2 * L)
    tw_col = jnp.concatenate([tw1, tw2], axis=0)
    w1a = expert_w1[:, :, :DFF].astype(BF)
    w1b = expert_w1[:, :, DFF:].astype(BF)
    w2b = expert_w2.astype(BF)
    es = _gmlp(sc, pos_row, tw_col, h2b, w1a, w1b, w2b)
    out = _combine(pos1, pos2, x2, es)
    return out.reshape(1, L, D), cnt.reshape(NE)


# speedup vs baseline: 1.3481x; 1.3481x over previous
import functools

import jax
import jax.numpy as jnp
from jax.experimental import pallas as pl
from jax.experimental.pallas import tpu as pltpu

L = 2048
D = 1024
NH = 16
HD = 64
DC = 128
DFF = 2048
NE = 8
EPS = 1.1920929e-07
NEG = -1e30
BF = jnp.bfloat16
F32 = jnp.float32
I32 = jnp.int32
BT = 256
NT = (2 * L) // BT + NE
PADT = NT * BT
HIGHEST = jax.lax.Precision.HIGHEST


def _dot(a, b):
    return jax.lax.dot_general(a.astype(BF), b.astype(BF),
                               (((a.ndim - 1,), (0,)), ((), ())),
                               preferred_element_type=F32)


def _dot_t(a, b):
    return jax.lax.dot_general(a.astype(BF), b.astype(BF),
                               (((1,), (1,)), ((), ())),
                               preferred_element_type=F32)


def _dot_f32(a, b):
    return jax.lax.dot_general(a, b, (((a.ndim - 1,), (0,)), ((), ())),
                               precision=HIGHEST,
                               preferred_element_type=F32)


def _qkv_body(x_ref, anw_ref, wkv_ref, wkc_ref, wvc_ref, wqr_ref, wkr_ref,
              q_ref, kc_ref, kr_ref, v_ref):
    x = x_ref[...]
    var = jnp.mean(x * x, axis=-1, keepdims=True)
    h = x * jax.lax.rsqrt(var + EPS) * anw_ref[...]
    scale = HD ** -0.5
    q_ref[...] = (_dot(h, wqr_ref[...]) * scale).astype(BF)
    c = _dot(h, wkv_ref[...])
    kc_ref[...] = _dot(c, wkc_ref[...]).astype(BF)
    kr_ref[...] = _dot(h, wkr_ref[...]).astype(BF)
    v_ref[...] = _dot(c, wvc_ref[...]).astype(BF)


def _qkv(x2d, anw, wkv, wkc, wvc, wqr, wkr):
    blk = 512
    w_spec = lambda shape: pl.BlockSpec(shape, lambda i: (0,) * len(shape))
    row = pl.BlockSpec((blk, D), lambda i: (i, 0))
    return pl.pallas_call(
        _qkv_body,
        grid=(L // blk,),
        in_specs=[row, w_spec((1, D)), w_spec((D, DC)), w_spec((DC, D)),
                  w_spec((DC, D)), w_spec((D, D)), w_spec((D, D))],
        out_specs=[row, row, row, row],
        out_shape=[jax.ShapeDtypeStruct((L, D), BF)] * 4,
    )(x2d, anw.reshape(1, D), wkv, wkc, wvc, wqr, wkr)


def _attn_body(q_ref, kc_ref, kr_ref, v_ref, o_ref):
    cb = 512
    for hh in range(2):
        sl = slice(hh * HD, (hh + 1) * HD)
        k2 = jnp.concatenate([kc_ref[:, sl], kr_ref[:, sl]], axis=1)
        v = v_ref[:, sl]
        for c0 in range(0, L, cb):
            q = q_ref[c0:c0 + cb, sl]
            q2 = jnp.concatenate([q, q], axis=1)
            s = _dot_t(q2, k2)
            m = jnp.max(s, axis=-1, keepdims=True)
            p = jnp.exp(s - m)
            denom = jnp.sum(p, axis=-1, keepdims=True)
            o = _dot((p / denom).astype(BF), v)
            o_ref[c0:c0 + cb, sl] = o.astype(BF)


def _attention(q, kc, kr, v):
    pair = pl.BlockSpec((L, 2 * HD), lambda i: (0, i))
    return pl.pallas_call(
        _attn_body,
        grid=(NH // 2,),
        in_specs=[pair, pair, pair, pair],
        out_specs=pair,
        out_shape=jax.ShapeDtypeStruct((L, D), BF),
    )(q, kc, kr, v)


def _post_body(attn_ref, x_ref, wo_ref, fnw_ref, gw_ref, gb_ref,
               x2_ref, h2_ref, e1_ref, e2_ref, tw1_ref, tw2_ref, cnt_ref):
    i = pl.program_id(0)
    x2 = _dot(attn_ref[...], wo_ref[...]) + x_ref[...]
    x2_ref[...] = x2
    var = jnp.mean(x2 * x2, axis=-1, keepdims=True)
    h2 = x2 * jax.lax.rsqrt(var + EPS) * fnw_ref[...]
    h2b = h2.astype(BF)
    h2_ref[...] = h2b
    logits = _dot(h2b, gw_ref[...]) + gb_ref[...]
    rows = logits.shape[0]
    iota = jax.lax.broadcasted_iota(I32, (rows, NE), 1)
    m1 = jnp.max(logits, axis=-1, keepdims=True)
    i1 = jnp.min(jnp.where(logits == m1, iota, NE), axis=-1, keepdims=True)
    l2 = jnp.where(iota == i1, NEG, logits)
    m2 = jnp.max(l2, axis=-1, keepdims=True)
    i2 = jnp.min(jnp.where(l2 == m2, iota, NE), axis=-1, keepdims=True)
    tw1 = 1.0 / (1.0 + jnp.exp(m2 - m1))
    e1_ref[...] = i1
    e2_ref[...] = i2
    tw1_ref[...] = tw1
    tw2_ref[...] = 1.0 - tw1
    oh = ((iota == i1) | (iota == i2)).astype(F32)
    cnt = jnp.sum(oh, axis=0, keepdims=True)

    @pl.when(i == 0)
    def _():
        cnt_ref[...] = jnp.zeros_like(cnt_ref)

    cnt_ref[...] += cnt


def _post(attn, x2d, wo, fnw, gw, gb):
    blk = 512
    w_spec = lambda shape: pl.BlockSpec(shape, lambda i: (0,) * len(shape))
    row = pl.BlockSpec((blk, D), lambda i: (i, 0))
    col = pl.BlockSpec((blk, 1), lambda i: (i, 0))
    return pl.pallas_call(
        _post_body,
        grid=(L // blk,),
        in_specs=[row, row, w_spec((D, D)), w_spec((1, D)), w_spec((D, NE)),
                  w_spec((1, NE))],
        out_specs=[row, row, col, col, col, col, w_spec((1, NE))],
        out_shape=[
            jax.ShapeDtypeStruct((L, D), F32),
            jax.ShapeDtypeStruct((L, D), BF),
            jax.ShapeDtypeStruct((L, 1), I32),
            jax.ShapeDtypeStruct((L, 1), I32),
            jax.ShapeDtypeStruct((L, 1), F32),
            jax.ShapeDtypeStruct((L, 1), F32),
            jax.ShapeDtypeStruct((1, NE), F32),
        ],
    )(attn, x2d, wo, fnw.reshape(1, D), gw, gb.reshape(1, NE))


def _meta_body(e1_ref, e2_ref, pos1_ref, pos2_ref, sc_ref):
    iota8 = jax.lax.broadcasted_iota(I32, (1, NE), 1)
    oh1 = (e1_ref[...] == iota8).astype(F32)
    oh2 = (e2_ref[...] == iota8).astype(F32)
    ch = 256
    ir = jax.lax.broadcasted_iota(I32, (ch, ch), 0)
    ic = jax.lax.broadcasted_iota(I32, (ch, ch), 1)
    tstrict = (ic < ir).astype(BF)
    carry = jnp.zeros((1, NE), F32)
    ranks = []
    for oh in (oh1, oh2):
        rs = []
        for c0 in range(0, L, ch):
            blk = oh[c0:c0 + ch, :]
            part = jax.lax.dot_general(
                tstrict, blk.astype(BF), (((1,), (0,)), ((), ())),
                preferred_element_type=F32)
            rs.append(part + carry)
            carry = carry + jnp.sum(blk, axis=0, keepdims=True)
        ranks.append(jnp.concatenate(rs, axis=0))
    counts = carry
    padded = jnp.ceil(counts / BT) * BT
    e8r = jax.lax.broadcasted_iota(I32, (NE, NE), 0)
    e8c = jax.lax.broadcasted_iota(I32, (NE, NE), 1)
    mupper = (e8r < e8c).astype(F32)
    off = _dot_f32(padded, mupper)
    pos1_ref[...] = jnp.sum(oh1 * (off + ranks[0]), axis=-1,
                            keepdims=True).astype(I32)
    pos2_ref[...] = jnp.sum(oh2 * (off + ranks[1]), axis=-1,
                            keepdims=True).astype(I32)
    tile_start = off / BT
    n_used = jnp.sum(padded) / BT
    jcol = jax.lax.broadcasted_iota(I32, (NT, 1), 0).astype(F32)
    jclamp = jnp.minimum(jcol, n_used - 1.0)
    emap = jnp.sum((tile_start <= jclamp).astype(F32), axis=-1,
                   keepdims=True) - 1.0
    active = (jcol < n_used).astype(F32)
    sc_ref[...] = jnp.concatenate([emap, active], axis=1).astype(I32)


def _meta(e1, e2):
    full = lambda shape: pl.BlockSpec(shape, lambda: (0,) * len(shape))
    return pl.pallas_call(
        _meta_body,
        in_specs=[full((L, 1)), full((L, 1))],
        out_specs=[full((L, 1)), full((L, 1)), full((NT, 2))],
        out_shape=[
            jax.ShapeDtypeStruct((L, 1), I32),
            jax.ShapeDtypeStruct((L, 1), I32),
            jax.ShapeDtypeStruct((NT, 2), I32),
        ],
    )(e1, e2)


def _silu(a):
    return a * (0.5 * (jnp.tanh(a * 0.5) + 1.0))


def _gmlp_body(sc_ref, posr_ref, twc_ref, h2_ref, w1a_ref, w1b_ref, w2_ref,
               es_ref):
    j = pl.program_id(0)
    base = j * BT

    @pl.when(sc_ref[j, 1] == 0)
    def _():
        es_ref[...] = jnp.zeros_like(es_ref)

    @pl.when(sc_ref[j, 1] == 1)
    def _():
        iota_r = jax.lax.broadcasted_iota(I32, (BT, 1), 0) + base
        a_sel = (iota_r == posr_ref[...]).astype(BF)
        pair_tok = jax.lax.broadcasted_iota(I32, (2 * L, 1), 0)
        pair_tok = jnp.where(pair_tok >= L, pair_tok - L, pair_tok)
        hi = (pair_tok // 8).astype(BF)
        lo = (pair_tok % 8).astype(BF)
        rhs = jnp.concatenate([hi, lo, twc_ref[...].astype(BF)], axis=1)
        hlw = jax.lax.dot_general(a_sel, rhs, (((1,), (0,)), ((), ())),
                                  preferred_element_type=F32)
        row_ids = hlw[:, 0:1] * 8.0 + hlw[:, 1:2]
        ws = hlw[:, 2:3]
        iota_c = jax.lax.broadcasted_iota(I32, (BT, L), 1).astype(F32)
        g_sel = (row_ids == iota_c).astype(BF)
        xs = jax.lax.dot_general(g_sel, h2_ref[...],
                                 (((1,), (0,)), ((), ())),
                                 preferred_element_type=F32)
        xsb = xs.astype(BF)
        a = _dot(xsb, w1a_ref[0])
        b = _dot(xsb, w1b_ref[0])
        g = (_silu(a) * b).astype(BF)
        eo = _dot(g, w2_ref[0])
        es_ref[...] = (eo * ws).astype(BF)


def _gmlp(sc, pos_row, tw_col, h2b, w1a, w1b, w2):
    grid_spec = pltpu.PrefetchScalarGridSpec(
        num_scalar_prefetch=1,
        grid=(NT,),
        in_specs=[
            pl.BlockSpec((1, 2 * L), lambda j, sc: (0, 0)),
            pl.BlockSpec((2 * L, 1), lambda j, sc: (0, 0)),
            pl.BlockSpec((L, D), lambda j, sc: (0, 0)),
            pl.BlockSpec((1, D, DFF), lambda j, sc: (sc[j, 0], 0, 0)),
            pl.BlockSpec((1, D, DFF), lambda j, sc: (sc[j, 0], 0, 0)),
            pl.BlockSpec((1, DFF, D), lambda j, sc: (sc[j, 0], 0, 0)),
        ],
        out_specs=pl.BlockSpec((BT, D), lambda j, sc: (j, 0)),
    )
    return pl.pallas_call(
        _gmlp_body,
        grid_spec=grid_spec,
        out_shape=jax.ShapeDtypeStruct((PADT, D), BF),
    )(sc, pos_row, tw_col, h2b, w1a, w1b, w2)


def _combine_body(pos1_ref, pos2_ref, x2_ref, es_ref, out_ref):
    iota_c = jax.lax.broadcasted_iota(I32, (BT, PADT), 1)
    w_sel = ((iota_c == pos1_ref[...]).astype(BF)
             + (iota_c == pos2_ref[...]).astype(BF))
    moe = jax.lax.dot_general(w_sel, es_ref[...], (((1,), (0,)), ((), ())),
                              preferred_element_type=F32)
    out_ref[...] = moe + x2_ref[...]


def _combine(pos1, pos2, x2, es):
    col = pl.BlockSpec((BT, 1), lambda i: (i, 0))
    row = pl.BlockSpec((BT, D), lambda i: (i, 0))
    full = lambda shape: pl.BlockSpec(shape, lambda i: (0,) * len(shape))
    return pl.pallas_call(
        _combine_body,
        grid=(L // BT,),
        in_specs=[col, col, row, full((PADT, D))],
        out_specs=row,
        out_shape=jax.ShapeDtypeStruct((L, D), F32),
    )(pos1, pos2, x2, es)


def kernel(x, attn_norm_w, ffn_norm_w, w_kv_c, w_kc_up, w_vc_up, w_qr, w_kr,
           w_o, gate_w, expert_bias, expert_w1, expert_w2):
    x2d = x.reshape(L, D)
    q, kc, kr, v = _qkv(x2d, attn_norm_w, w_kv_c, w_kc_up, w_vc_up, w_qr,
                        w_kr)
    attn = _attention(q, kc, kr, v)
    x2, h2b, e1, e2, tw1, tw2, cnt = _post(attn, x2d, w_o, ffn_norm_w,
                                           gate_w, expert_bias)
    pos1, pos2, sc = _meta(e1, e2)
    pos_row = jnp.concatenate([pos1, pos2], axis=0).reshape(1, 2 * L)
    tw_col = jnp.concatenate([tw1, tw2], axis=0)
    w1a = expert_w1[:, :, :DFF].astype(BF)
    w1b = expert_w1[:, :, DFF:].astype(BF)
    w2b = expert_w2.astype(BF)
    es = _gmlp(sc, pos_row, tw_col, h2b, w1a, w1b, w2b)
    out = _combine(pos1, pos2, x2, es)
    return out.reshape(1, L, D), cnt.reshape(NE)
